# baseline (speedup 1.0000x reference)
import functools
import jax
import jax.numpy as jnp
from jax import lax
from jax.experimental import pallas as pl
from jax.experimental.pallas import tpu as pltpu
from jax.experimental.pallas import tpu_sc as plsc

_NC = 2
_NS = 16
_NW = _NC * _NS


def kernel(src, table):
    seq_len, batch = src.shape
    max_len, hidden = table.shape
    flat_cols = batch * hidden

    rows_per_w = seq_len // _NW
    chunk = 64
    n_chunks = rows_per_w // chunk

    mesh = plsc.VectorSubcoreMesh(core_axis_name="c", subcore_axis_name="s")

    @functools.partial(
        pl.kernel,
        mesh=mesh,
        out_type=jax.ShapeDtypeStruct((seq_len, flat_cols), jnp.float32),
        scratch_types=[
            pltpu.VMEM((chunk, hidden), jnp.float32),
        ],
    )
    def k(table_hbm, out_hbm, buf):
        c = lax.axis_index("c")
        s = lax.axis_index("s")
        wid = s * _NC + c
        base = wid * rows_per_w

        def body(j, carry):
            r0 = base + j * chunk
            pltpu.sync_copy(table_hbm.at[pl.ds(r0, chunk)], buf)

            @pl.when(jnp.logical_and(wid == 0, j == 0))
            def _():
                def zb(i, c2):
                    buf[0, pl.ds(i * 16, 16)] = jnp.zeros((16,), jnp.float32)
                    return c2
                lax.fori_loop(0, hidden // 16, zb, 0)

            for b in range(batch):
                pltpu.sync_copy(
                    buf, out_hbm.at[pl.ds(r0, chunk), pl.ds(b * hidden, hidden)]
                )
            return carry

        lax.fori_loop(0, n_chunks, body, 0)

    out = k(table)
    return out.reshape(seq_len, batch, hidden)
